# baseline (speedup 1.0000x reference)
import functools

import jax
import jax.numpy as jnp
from jax import lax
from jax.experimental import pallas as pl
from jax.experimental.pallas import tpu as pltpu
from jax.experimental.pallas import tpu_sc as plsc

_SIZE = 3
_COLS = 1024
_NW = 32
_CPW = _COLS // _NW
_LANES = 16

_mesh = plsc.VectorSubcoreMesh(core_axis_name="c", subcore_axis_name="s")


@functools.partial(
    pl.kernel,
    mesh=_mesh,
    out_type=jax.ShapeDtypeStruct((_SIZE, _COLS), jnp.float32),
    scratch_types=[
        pltpu.VMEM((_SIZE, _SIZE, _LANES), jnp.float32),
        pltpu.VMEM((_SIZE, _CPW), jnp.float32),
        pltpu.VMEM((_SIZE, _CPW), jnp.float32),
        pltpu.SemaphoreType.DMA,
    ],
)
def _spmv(v_hbm, m_hbm, out_hbm, m_v, v_v, o_v, sem):
    wid = lax.axis_index("s") * 2 + lax.axis_index("c")
    base = wid * _CPW
    copies = [pltpu.async_copy(m_hbm, m_v, sem)]
    for k in range(_SIZE):
        copies.append(
            pltpu.async_copy(
                v_hbm.at[k, pl.ds(base, _CPW)], v_v.at[k], sem
            )
        )
    for c in copies:
        c.wait()
    m = [[m_v[r, k, :] for k in range(_SIZE)] for r in range(_SIZE)]
    for j in range(_CPW // _LANES):
        sl = pl.ds(j * _LANES, _LANES)
        rows = [v_v[k, sl] for k in range(_SIZE)]
        for r in range(_SIZE):
            acc = m[r][0] * rows[0]
            for k in range(1, _SIZE):
                acc = acc + m[r][k] * rows[k]
            o_v[r, sl] = acc
    out_copies = [
        pltpu.async_copy(o_v.at[r], out_hbm.at[r, pl.ds(base, _CPW)], sem)
        for r in range(_SIZE)
    ]
    for c in out_copies:
        c.wait()


def kernel(v, M_hat):
    m_b = jnp.broadcast_to(M_hat[:, :, None], (_SIZE, _SIZE, _LANES))
    return _spmv(v, m_b)


# speedup vs baseline: 14.5238x; 14.5238x over previous
import jax
import jax.numpy as jnp
from jax.experimental import pallas as pl
from jax.experimental.pallas import tpu as pltpu

_SIZE = 3
_COLS = 1024


def _body(m_ref, v_ref, o_ref):
    m = m_ref[...]
    v = v_ref[...]
    acc = m[:, 0:1] * v[0:1, :]
    for k in range(1, _SIZE):
        acc = acc + m[:, k:k + 1] * v[k:k + 1, :]
    o_ref[...] = acc


def kernel(v, M_hat):
    return pl.pallas_call(
        _body,
        out_shape=jax.ShapeDtypeStruct((_SIZE, _COLS), jnp.float32),
        in_specs=[
            pl.BlockSpec(memory_space=pltpu.VMEM),
            pl.BlockSpec(memory_space=pltpu.VMEM),
        ],
        out_specs=pl.BlockSpec(memory_space=pltpu.VMEM),
    )(M_hat, v)
